# baseline (speedup 1.0000x reference)
import functools
import math

import jax
import jax.numpy as jnp
from jax import lax
from jax.experimental import pallas as pl
from jax.experimental.pallas import tpu as pltpu
from jax.experimental.pallas import tpu_sc as plsc

DIM = 64
SCALE = math.sqrt(DIM)
LANES = 16
NC, NS = 2, 16
NW = NC * NS

G = 128
K = 4
C = K * G


def _make_kernel(B):
    assert B % (NW * C) == 0
    n_chunks = B // (NW * C)
    xrows_per_chunk = K
    mesh = plsc.VectorSubcoreMesh(core_axis_name="c", subcore_axis_name="s")

    @functools.partial(
        pl.kernel,
        out_type=jax.ShapeDtypeStruct((B, DIM), jnp.float32),
        mesh=mesh,
        scratch_types=[
            pltpu.VMEM((K, G), jnp.int32),
            pltpu.VMEM((C, DIM), jnp.float32),
            pltpu.SemaphoreType.DMA,
        ],
        compiler_params=pltpu.CompilerParams(use_tc_tiling_on_sc=False),
    )
    def emb_kernel(x_hbm, w_hbm, out_hbm, idx_v, rows_v, sem):
        wid = lax.axis_index("s") * NC + lax.axis_index("c")

        def chunk_body(ci, _):
            xrow = (wid * n_chunks + ci) * xrows_per_chunk
            base = (wid * n_chunks + ci) * C
            pltpu.sync_copy(x_hbm.at[pl.ds(xrow, xrows_per_chunk)], idx_v)
            copies = [
                pltpu.async_copy(
                    w_hbm.at[idx_v.at[j]], rows_v.at[pl.ds(j * G, G)], sem
                )
                for j in range(K)
            ]
            for cp in copies:
                cp.wait()

            def scale_body(r):
                for j in range(DIM // LANES):
                    sl = pl.ds(j * LANES, LANES)
                    rows_v[r, sl] = rows_v[r, sl] * SCALE

            plsc.parallel_loop(0, C, 1, unroll=8)(scale_body)
            pltpu.sync_copy(rows_v, out_hbm.at[pl.ds(base, C)])
            return ()

        lax.fori_loop(0, n_chunks, chunk_body, ())

    return emb_kernel


@jax.jit
def kernel(x, W):
    B = x.shape[0] * x.shape[1]
    xf = x.reshape(B // G, G)
    out = _make_kernel(B)(xf, W)
    return out.reshape(x.shape[0], x.shape[1], DIM)


# speedup vs baseline: 1.0738x; 1.0738x over previous
import functools
import math

import jax
import jax.numpy as jnp
from jax import lax
from jax.experimental import pallas as pl
from jax.experimental.pallas import tpu as pltpu
from jax.experimental.pallas import tpu_sc as plsc

DIM = 64
SCALE = math.sqrt(DIM)
LANES = 16
NC, NS = 2, 16
NW = NC * NS

G = 128
K = 4
C = K * G
NBUF = 2


def _make_kernel(B):
    assert B % (NW * C * NBUF) == 0
    n_chunks = B // (NW * C)
    mesh = plsc.VectorSubcoreMesh(core_axis_name="c", subcore_axis_name="s")

    @functools.partial(
        pl.kernel,
        out_type=jax.ShapeDtypeStruct((B, DIM), jnp.float32),
        mesh=mesh,
        scratch_types=[
            pltpu.VMEM((NBUF, K, G), jnp.int32),
            pltpu.VMEM((NBUF, C, DIM), jnp.float32),
            [pltpu.SemaphoreType.DMA] * NBUF,
            [pltpu.SemaphoreType.DMA] * NBUF,
        ],
        compiler_params=pltpu.CompilerParams(use_tc_tiling_on_sc=False),
    )
    def emb_kernel(x_hbm, w_hbm, out_hbm, idx_v, rows_v, gsem, ssem):
        wid = lax.axis_index("s") * NC + lax.axis_index("c")
        chunk0 = wid * n_chunks

        def fire_gathers(ci, b):
            pltpu.sync_copy(x_hbm.at[pl.ds((chunk0 + ci) * K, K)], idx_v.at[b])
            for j in range(K):
                pltpu.async_copy(
                    w_hbm.at[idx_v.at[b].at[j]],
                    rows_v.at[b].at[pl.ds(j * G, G)],
                    gsem[b],
                )

        def wait_gathers(b):
            for j in range(K):
                pltpu.make_async_copy(
                    w_hbm.at[idx_v.at[b].at[j]],
                    rows_v.at[b].at[pl.ds(j * G, G)],
                    gsem[b],
                ).wait()

        def fire_store(ci, b):
            pltpu.async_copy(
                rows_v.at[b], out_hbm.at[pl.ds((chunk0 + ci) * C, C)], ssem[b]
            )

        def wait_store(ci, b):
            pltpu.make_async_copy(
                rows_v.at[b], out_hbm.at[pl.ds((chunk0 + ci) * C, C)], ssem[b]
            ).wait()

        def scale(b):
            def scale_body(r):
                for j in range(DIM // LANES):
                    sl = pl.ds(j * LANES, LANES)
                    rows_v[b, r, sl] = rows_v[b, r, sl] * SCALE

            plsc.parallel_loop(0, C, 1, unroll=8)(scale_body)

        fire_gathers(0, 0)

        def super_body(s, _):
            for b in range(NBUF):
                ci = s * NBUF + b
                nci = ci + 1
                nb = (b + 1) % NBUF

                @pl.when(jnp.logical_and(nci >= NBUF, nci < n_chunks))
                def _():
                    wait_store(nci - NBUF, nb)

                @pl.when(nci < n_chunks)
                def _():
                    fire_gathers(nci, nb)

                wait_gathers(b)
                scale(b)
                fire_store(ci, b)
            return ()

        lax.fori_loop(0, n_chunks // NBUF, super_body, ())

        for b in range(NBUF):
            wait_store(n_chunks - NBUF + b, b)

    return emb_kernel


@jax.jit
def kernel(x, W):
    B = x.shape[0] * x.shape[1]
    xf = x.reshape(B // G, G)
    out = _make_kernel(B)(xf, W)
    return out.reshape(x.shape[0], x.shape[1], DIM)
